# baseline (speedup 1.0000x reference)
import functools

import jax
import jax.numpy as jnp
from jax import lax
from jax.experimental import pallas as pl
from jax.experimental.pallas import tpu as pltpu
from jax.experimental.pallas import tpu_sc as plsc

ALIGNED_H = 7
ALIGNED_W = 7
SPATIAL_SCALE = 0.25

_NC = 2
_NS = 16
_L = 16
_NW = _NC * _NS


@functools.cache
def _make_sc_blend(S, C, G):
    ROWS = 4 * G
    WPW = S // (_NW * G)
    mesh = plsc.VectorSubcoreMesh(core_axis_name="c", subcore_axis_name="s")

    @functools.partial(
        pl.kernel,
        out_type=jax.ShapeDtypeStruct((S, C), jnp.float32),
        mesh=mesh,
        scratch_types=[
            pltpu.VMEM((ROWS,), jnp.int32),
            pltpu.VMEM((ROWS, C), jnp.float32),
            pltpu.VMEM((G, 4, _L), jnp.float32),
            pltpu.VMEM((G, C), jnp.float32),
            pltpu.SemaphoreType.DMA,
        ],
    )
    def sc_blend(table_hbm, idx_hbm, w_hbm, out_hbm, idx_v, rows_v, w_v, out_v, sem):
        wid = lax.axis_index("s") * _NC + lax.axis_index("c")

        @pl.loop(0, WPW)
        def _win(t):
            base = (wid * WPW + t) * G
            pltpu.sync_copy(idx_hbm.at[pl.ds(base * 4, ROWS)], idx_v)
            pltpu.sync_copy(w_hbm.at[pl.ds(base, G)], w_v)
            pltpu.async_copy(table_hbm.at[idx_v], rows_v, sem).wait()

            @pl.loop(0, G)
            def _samp(i):
                r = i * 4
                w0 = w_v[i, 0, :]
                w1 = w_v[i, 1, :]
                w2 = w_v[i, 2, :]
                w3 = w_v[i, 3, :]
                for cc in range(C // _L):
                    sl = pl.ds(cc * _L, _L)
                    out_v[i, sl] = (rows_v[r, sl] * w0 + rows_v[r + 1, sl] * w1
                                    + rows_v[r + 2, sl] * w2 + rows_v[r + 3, sl] * w3)

            pltpu.sync_copy(out_v, out_hbm.at[pl.ds(base, G)])

    return sc_blend


def _prep(features, rois):
    B, C, H, W = features.shape
    N = rois.shape[0]
    AH, AW = ALIGNED_H, ALIGNED_W
    batch_idx = rois[:, 0].astype(jnp.int32)
    x1 = rois[:, 1] * SPATIAL_SCALE
    y1 = rois[:, 2] * SPATIAL_SCALE
    x2 = rois[:, 3] * SPATIAL_SCALE
    y2 = rois[:, 4] * SPATIAL_SCALE
    roi_w = jnp.maximum(x2 - x1, 0.0)
    roi_h = jnp.maximum(y2 - y1, 0.0)
    bin_w = roi_w / float(AW - 1)
    bin_h = roi_h / float(AH - 1)
    ph = jnp.arange(AH, dtype=jnp.float32)
    pw = jnp.arange(AW, dtype=jnp.float32)
    h = y1[:, None] + ph[None, :] * bin_h[:, None]
    w = x1[:, None] + pw[None, :] * bin_w[:, None]
    valid_h = (h >= 0) & (h < H)
    valid_w = (w >= 0) & (w < W)
    hs = jnp.minimum(jnp.floor(h), H - 2)
    ws = jnp.minimum(jnp.floor(w), W - 2)
    hs_i = jnp.clip(hs.astype(jnp.int32), 0, H - 2)
    ws_i = jnp.clip(ws.astype(jnp.int32), 0, W - 2)
    h_ratio = h - hs_i.astype(jnp.float32)
    w_ratio = w - ws_i.astype(jnp.float32)

    valid = (valid_h[:, :, None] & valid_w[:, None, :]).astype(jnp.float32)
    hr = h_ratio[:, :, None]
    wr = w_ratio[:, None, :]
    w4 = jnp.stack(
        [(1.0 - hr) * (1.0 - wr) * valid,
         (1.0 - hr) * wr * valid,
         hr * (1.0 - wr) * valid,
         hr * wr * valid],
        axis=-1,
    ).reshape(N * AH * AW, 4)
    tl = (batch_idx[:, None, None] * (H * W)
          + hs_i[:, :, None] * W + ws_i[:, None, :])
    idx4 = jnp.stack([tl, tl + 1, tl + W, tl + W + 1], axis=-1)
    idx4 = idx4.reshape(N * AH * AW * 4).astype(jnp.int32)
    return idx4, w4


def kernel(features, rois):
    B, C, H, W = features.shape
    N = rois.shape[0]
    AH, AW = ALIGNED_H, ALIGNED_W
    S = N * AH * AW
    G = 32
    table = jnp.transpose(features, (0, 2, 3, 1)).reshape(B * H * W, C)
    idx4, w4 = _prep(features, rois)

    pad = (-S) % (_NW * G)
    if pad:
        idx4 = jnp.concatenate([idx4, jnp.zeros((pad * 4,), jnp.int32)])
        w4 = jnp.concatenate([w4, jnp.zeros((pad, 4), jnp.float32)])
    wb = jnp.broadcast_to(w4[:, :, None], (S + pad, 4, _L))

    out = _make_sc_blend(S + pad, C, G)(table, idx4, wb)
    out = out[:S].reshape(N, AH, AW, C)
    return jnp.transpose(out, (0, 3, 1, 2))


# speedup vs baseline: 1.5230x; 1.5230x over previous
import functools

import jax
import jax.numpy as jnp
from jax import lax
from jax.experimental import pallas as pl
from jax.experimental.pallas import tpu as pltpu
from jax.experimental.pallas import tpu_sc as plsc

ALIGNED_H = 7
ALIGNED_W = 7
SPATIAL_SCALE = 0.25

_NC = 2
_NS = 16
_L = 16
_NW = _NC * _NS


@functools.cache
def _make_sc_blend(S, C, G):
    ROWS = 4 * G
    WPW = S // (_NW * G)
    assert WPW % 4 == 0 and WPW >= 8
    mesh = plsc.VectorSubcoreMesh(core_axis_name="c", subcore_axis_name="s")

    @functools.partial(
        pl.kernel,
        out_type=jax.ShapeDtypeStruct((S, C), jnp.float32),
        mesh=mesh,
        scratch_types=[
            pltpu.VMEM((4, ROWS), jnp.int32),
            pltpu.VMEM((2, ROWS, C), jnp.float32),
            pltpu.VMEM((4, G, 4, _L), jnp.float32),
            pltpu.VMEM((2, G, C), jnp.float32),
        ] + [pltpu.SemaphoreType.DMA] * 12,
    )
    def sc_blend(table_hbm, idx_hbm, w_hbm, out_hbm,
                 idx_v, rows_v, w_v, out_v, *sems):
        isem, wsem, gsem, osem = sems[0:4], sems[4:8], sems[8:10], sems[10:12]
        wid = lax.axis_index("s") * _NC + lax.axis_index("c")

        def idx_copy(u, s):
            return pltpu.make_async_copy(
                idx_hbm.at[pl.ds((wid * WPW + u) * ROWS, ROWS)],
                idx_v.at[s], isem[s])

        def w_copy(u, s):
            return pltpu.make_async_copy(
                w_hbm.at[pl.ds((wid * WPW + u) * G, G)], w_v.at[s], wsem[s])

        def g_copy(isl, rsl):
            return pltpu.make_async_copy(
                table_hbm.at[idx_v.at[isl]], rows_v.at[rsl], gsem[rsl])

        def o_copy(u, s):
            return pltpu.make_async_copy(
                out_v.at[s], out_hbm.at[pl.ds((wid * WPW + u) * G, G)], osem[s])

        def blend(rsl, wsl):
            @plsc.parallel_loop(0, G, step=1, unroll=2)
            def _samp(i):
                r = i * 4
                w0 = w_v[wsl, i, 0, :]
                w1 = w_v[wsl, i, 1, :]
                w2 = w_v[wsl, i, 2, :]
                w3 = w_v[wsl, i, 3, :]
                for cc in range(C // _L):
                    sl = pl.ds(cc * _L, _L)
                    a = rows_v[rsl, r, sl] * w0 + rows_v[rsl, r + 1, sl] * w1
                    b = rows_v[rsl, r + 2, sl] * w2 + rows_v[rsl, r + 3, sl] * w3
                    out_v[rsl, i, sl] = a + b

        for h in range(4):
            idx_copy(h, h).start()
            w_copy(h, h).start()
        idx_copy(0, 0).wait()
        g_copy(0, 0).start()

        @pl.loop(0, WPW, step=4)
        def _win(t):
            for h in range(4):
                u = t + h
                rsl = h % 2
                g_copy(h, rsl).wait()
                @pl.when(u + 4 < WPW)
                def _():
                    idx_copy(u + 4, h).start()
                nsl = (h + 1) % 4
                if h == 3:
                    @pl.when(t + 4 < WPW)
                    def _():
                        idx_copy(0, nsl).wait()
                        w_copy(0, h).wait()
                        g_copy(nsl, (h + 1) % 2).start()
                    @pl.when(t + 4 >= WPW)
                    def _():
                        w_copy(0, h).wait()
                else:
                    idx_copy(0, nsl).wait()
                    w_copy(0, h).wait()
                    g_copy(nsl, (h + 1) % 2).start()
                if h < 2:
                    @pl.when(u >= 2)
                    def _():
                        o_copy(0, rsl).wait()
                else:
                    o_copy(0, rsl).wait()
                blend(rsl, h)
                o_copy(u, rsl).start()
                @pl.when(u + 4 < WPW)
                def _():
                    w_copy(u + 4, h).start()

        o_copy(0, 0).wait()
        o_copy(0, 1).wait()

    return sc_blend


def _prep(features, rois):
    B, C, H, W = features.shape
    N = rois.shape[0]
    AH, AW = ALIGNED_H, ALIGNED_W
    batch_idx = rois[:, 0].astype(jnp.int32)
    x1 = rois[:, 1] * SPATIAL_SCALE
    y1 = rois[:, 2] * SPATIAL_SCALE
    x2 = rois[:, 3] * SPATIAL_SCALE
    y2 = rois[:, 4] * SPATIAL_SCALE
    roi_w = jnp.maximum(x2 - x1, 0.0)
    roi_h = jnp.maximum(y2 - y1, 0.0)
    bin_w = roi_w / float(AW - 1)
    bin_h = roi_h / float(AH - 1)
    ph = jnp.arange(AH, dtype=jnp.float32)
    pw = jnp.arange(AW, dtype=jnp.float32)
    h = y1[:, None] + ph[None, :] * bin_h[:, None]
    w = x1[:, None] + pw[None, :] * bin_w[:, None]
    valid_h = (h >= 0) & (h < H)
    valid_w = (w >= 0) & (w < W)
    hs = jnp.minimum(jnp.floor(h), H - 2)
    ws = jnp.minimum(jnp.floor(w), W - 2)
    hs_i = jnp.clip(hs.astype(jnp.int32), 0, H - 2)
    ws_i = jnp.clip(ws.astype(jnp.int32), 0, W - 2)
    h_ratio = h - hs_i.astype(jnp.float32)
    w_ratio = w - ws_i.astype(jnp.float32)

    valid = (valid_h[:, :, None] & valid_w[:, None, :]).astype(jnp.float32)
    hr = h_ratio[:, :, None]
    wr = w_ratio[:, None, :]
    w4 = jnp.stack(
        [(1.0 - hr) * (1.0 - wr) * valid,
         (1.0 - hr) * wr * valid,
         hr * (1.0 - wr) * valid,
         hr * wr * valid],
        axis=-1,
    ).reshape(N * AH * AW, 4)
    tl = (batch_idx[:, None, None] * (H * W)
          + hs_i[:, :, None] * W + ws_i[:, None, :])
    idx4 = jnp.stack([tl, tl + 1, tl + W, tl + W + 1], axis=-1)
    idx4 = idx4.reshape(N * AH * AW * 4).astype(jnp.int32)
    return idx4, w4


def kernel(features, rois):
    B, C, H, W = features.shape
    N = rois.shape[0]
    AH, AW = ALIGNED_H, ALIGNED_W
    S = N * AH * AW
    G = 16
    table = jnp.transpose(features, (0, 2, 3, 1)).reshape(B * H * W, C)
    idx4, w4 = _prep(features, rois)

    pad = (-S) % (_NW * G * 4)
    if pad:
        idx4 = jnp.concatenate([idx4, jnp.zeros((pad * 4,), jnp.int32)])
        w4 = jnp.concatenate([w4, jnp.zeros((pad, 4), jnp.float32)])
    wb = jnp.broadcast_to(w4[:, :, None], (S + pad, 4, _L))

    out = _make_sc_blend(S + pad, C, G)(table, idx4, wb)
    out = out[:S].reshape(N, AH, AW, C)
    return jnp.transpose(out, (0, 3, 1, 2))
